# baseline (speedup 1.0000x reference)
import functools

import jax
import jax.numpy as jnp
from jax import lax
from jax.experimental import pallas as pl
from jax.experimental.pallas import tpu as pltpu
from jax.experimental.pallas import tpu_sc as plsc

_NUM_ENT = 50000
_NF = 3
_GCN = 128
_B = 4096

_NC, _NS = 2, 16
_NW = _NC * _NS
_CH = 128

_N_SUB, _N_OBJ, _N_YP, _N_REL = 3, 9, 3, 3
_N_TOT = _N_SUB + _N_OBJ + _N_YP + _N_REL


def _build_gather_indices(sub, rel, obj):
    r3 = jnp.arange(3, dtype=jnp.int32)
    idx_sub = (3 * sub[:, None] + r3).reshape(_NW, _N_SUB, _CH)
    idx_obj = (3 * jnp.repeat(obj, 3)[:, None] + r3).reshape(_NW, _N_OBJ, _CH)
    perms = [
        jax.random.permutation(jax.random.fold_in(jax.random.key(123), c), _B)
        for c in range(3)
    ]
    j_of = (1, 2, 2)
    idx_yp = jnp.concatenate(
        [3 * jnp.take(sub, perms[t]) + j_of[t] for t in range(3)]
    ).reshape(_NW, _N_YP, _CH)
    idx_rel = jnp.repeat(rel, 3).reshape(_NW, _N_REL, _CH)
    idx = jnp.concatenate([idx_sub, idx_obj, idx_yp, idx_rel], axis=1)
    return idx.astype(jnp.int32)


@functools.lru_cache(maxsize=1)
def _make_sc_gather():
    return functools.partial(
        pl.kernel,
        out_type=(
            jax.ShapeDtypeStruct((_NW * _N_SUB * _CH, _GCN), jnp.float32),
            jax.ShapeDtypeStruct((_NW * _N_OBJ * _CH, _GCN), jnp.float32),
            jax.ShapeDtypeStruct((_NW * _N_YP * _CH, _GCN), jnp.float32),
            jax.ShapeDtypeStruct((_NW * _N_REL * _CH, _GCN), jnp.float32),
        ),
        mesh=plsc.VectorSubcoreMesh(
            core_axis_name="c", subcore_axis_name="s",
            num_cores=_NC, num_subcores=_NS
        ),
        scratch_types=[
            pltpu.VMEM((_N_TOT, _CH), jnp.int32),
            pltpu.VMEM((_CH, _GCN), jnp.float32),
            pltpu.VMEM((_CH, _GCN), jnp.float32),
            pltpu.SemaphoreType.DMA,
            pltpu.SemaphoreType.DMA,
        ],
    )(_sc_gather_body)


def _sc_gather_body(xr, relt, idx, out_sub, out_obj, out_yp, out_rel,
                    idx_v, buf_a, buf_b, sem_a, sem_b):
    w = lax.axis_index("s") * _NC + lax.axis_index("c")
    pltpu.sync_copy(idx.at[w], idx_v)

    plan = [
        (xr, out_sub, _N_SUB, 0),
        (xr, out_obj, _N_OBJ, _N_SUB),
        (xr, out_yp, _N_YP, _N_SUB + _N_OBJ),
        (relt, out_rel, _N_REL, _N_SUB + _N_OBJ + _N_YP),
    ]
    chunks = []
    for tbl, out, n, c0 in plan:
        for c in range(n):
            chunks.append((tbl, out, n, c, c0 + c))

    bufs = (buf_a, buf_b)
    sems = (sem_a, sem_b)

    def start(g):
        tbl = chunks[g][0]
        cl = chunks[g][4]
        return pltpu.async_copy(tbl.at[idx_v.at[cl]], bufs[g % 2], sems[g % 2])

    h = start(0)
    for g in range(_N_TOT):
        h_next = start(g + 1) if g + 1 < _N_TOT else None
        h.wait()
        _, out, n, c, _ = chunks[g]
        dst = (w * n + c) * _CH
        pltpu.sync_copy(bufs[g % 2], out.at[pl.ds(dst, _CH)])
        h = h_next


def _mi_body(sub_ref, yp_ref, w1_ref, b1_ref, w2_ref, b2_ref,
             w3_ref, b3_ref, w4_ref, b4_ref, out_ref):
    hp = jax.lax.Precision.HIGHEST
    acc = jnp.float32(0.0)
    pairs = ((0, 1), (0, 2), (1, 2))
    for cnt, (i, j) in enumerate(pairs):
        xi = sub_ref[:, _GCN * i:_GCN * (i + 1)]
        yj = sub_ref[:, _GCN * j:_GCN * (j + 1)]
        ypc = yp_ref[cnt]
        h1 = jnp.maximum(
            jnp.dot(xi, w1_ref[cnt], precision=hp,
                    preferred_element_type=jnp.float32)
            + b1_ref[cnt:cnt + 1, :], 0.0)
        mu = (jnp.dot(h1, w2_ref[cnt], precision=hp,
                      preferred_element_type=jnp.float32)
              + b2_ref[cnt:cnt + 1, :])
        h2 = jnp.maximum(
            jnp.dot(xi, w3_ref[cnt], precision=hp,
                    preferred_element_type=jnp.float32)
            + b3_ref[cnt:cnt + 1, :], 0.0)
        logvar = jnp.tanh(
            jnp.dot(h2, w4_ref[cnt], precision=hp,
                    preferred_element_type=jnp.float32)
            + b4_ref[cnt:cnt + 1, :])
        inv_var = jnp.exp(-logvar)
        d = ((mu - ypc) ** 2 - (mu - yj) ** 2) * inv_var
        acc = acc + jnp.sum(d)
    out_ref[...] = (acc / jnp.float32(2 * _B)).reshape(1, 1)


def kernel(init_embed, init_rel, w_mu1, b_mu1, w_mu2, b_mu2,
           w_lv1, b_lv1, w_lv2, b_lv2, sub, rel, obj):
    xr = init_embed.reshape(_NUM_ENT * _NF, _GCN)
    idx = _build_gather_indices(sub, rel, obj)

    out_sub, out_obj, out_yp, out_rel = _make_sc_gather()(xr, init_rel, idx)

    sub_emb = out_sub.reshape(_B, _NF * _GCN)
    obj_emb = out_obj.reshape(_NF * _B, _NF * _GCN)
    rel_emb = out_rel.reshape(_B, _NF * _GCN)
    yp = out_yp.reshape(3, _B, _GCN)

    mi = pl.pallas_call(
        _mi_body,
        out_shape=jax.ShapeDtypeStruct((1, 1), jnp.float32),
    )(sub_emb, yp, w_mu1, b_mu1, w_mu2, b_mu2, w_lv1, b_lv1, w_lv2, b_lv2)
    mi_loss = mi[0, 0]

    return (sub_emb, rel_emb, obj_emb, init_embed, mi_loss)


# speedup vs baseline: 1.8188x; 1.8188x over previous
import functools

import jax
import jax.numpy as jnp
from jax import lax
from jax.experimental import pallas as pl
from jax.experimental.pallas import tpu as pltpu
from jax.experimental.pallas import tpu_sc as plsc

_NUM_ENT = 50000
_NF = 3
_GCN = 128
_DIM = _NF * _GCN
_B = 4096

_NC, _NS = 2, 16
_NW = _NC * _NS
_CH = _B // _NW

_J_OF = (1, 2, 2)


def _build_indices(sub, rel, obj):
    idx = jnp.stack(
        [sub.reshape(_NW, _CH), obj.reshape(_NW, _CH), rel.reshape(_NW, _CH)],
        axis=1,
    ).astype(jnp.int32)

    base = jnp.arange(_B, dtype=jnp.int32)
    obj_dst = jnp.stack([3 * base + t for t in range(3)])
    pinv = [
        jnp.argsort(
            jax.random.permutation(jax.random.fold_in(jax.random.key(123), c), _B)
        ).astype(jnp.int32)
        for c in range(3)
    ]
    yp_dst = jnp.stack([t * _B + pinv[t] for t in range(3)])
    dst = jnp.concatenate(
        [
            obj_dst.reshape(3, _NW, _CH).transpose(1, 0, 2),
            yp_dst.reshape(3, _NW, _CH).transpose(1, 0, 2),
        ],
        axis=1,
    ).astype(jnp.int32)
    return idx, dst


def _sc_gather_body(emb, relt, idx, dst, out_sub, out_obj, out_yp, out_rel,
                    idx_v, dst_v, buf_sub, buf_obj, buf_rel,
                    sem_a, sem_b, sem_c):
    w = lax.axis_index("s") * _NC + lax.axis_index("c")
    pltpu.sync_copy(idx.at[w], idx_v)
    pltpu.sync_copy(dst.at[w], dst_v)

    h_sub = pltpu.async_copy(emb.at[idx_v.at[0]], buf_sub, sem_a)
    h_obj = pltpu.async_copy(emb.at[idx_v.at[1]], buf_obj, sem_b)
    h_rel = pltpu.async_copy(relt.at[idx_v.at[2]], buf_rel, sem_c)

    h_sub.wait()
    pltpu.sync_copy(buf_sub, out_sub.at[pl.ds(w * _CH, _CH)])
    for t in range(3):
        j = _J_OF[t]
        pltpu.async_copy(
            buf_sub.at[:, pl.ds(j * _GCN, _GCN)],
            out_yp.at[dst_v.at[3 + t]],
            sem_a,
        ).wait()

    h_obj.wait()
    for t in range(3):
        pltpu.async_copy(buf_obj, out_obj.at[dst_v.at[t]], sem_b).wait()

    h_rel.wait()
    for t in range(3):
        pltpu.sync_copy(
            buf_rel,
            out_rel.at[pl.ds(w * _CH, _CH), pl.ds(t * _GCN, _GCN)],
        )


@functools.lru_cache(maxsize=1)
def _make_sc_gather():
    return functools.partial(
        pl.kernel,
        out_type=(
            jax.ShapeDtypeStruct((_B, _DIM), jnp.float32),
            jax.ShapeDtypeStruct((_NF * _B, _DIM), jnp.float32),
            jax.ShapeDtypeStruct((3 * _B, _GCN), jnp.float32),
            jax.ShapeDtypeStruct((_B, _DIM), jnp.float32),
        ),
        mesh=plsc.VectorSubcoreMesh(
            core_axis_name="c", subcore_axis_name="s",
            num_cores=_NC, num_subcores=_NS,
        ),
        scratch_types=[
            pltpu.VMEM((3, _CH), jnp.int32),
            pltpu.VMEM((6, _CH), jnp.int32),
            pltpu.VMEM((_CH, _DIM), jnp.float32),
            pltpu.VMEM((_CH, _DIM), jnp.float32),
            pltpu.VMEM((_CH, _GCN), jnp.float32),
            pltpu.SemaphoreType.DMA,
            pltpu.SemaphoreType.DMA,
            pltpu.SemaphoreType.DMA,
        ],
    )(_sc_gather_body)


def _mi_body(sub_ref, yp_ref, w1_ref, b1_ref, w2_ref, b2_ref,
             w3_ref, b3_ref, w4_ref, b4_ref, out_ref):
    hp = None
    mi = jnp.float32(0.0)
    pairs = ((0, 1), (0, 2), (1, 2))
    for cnt, (i, j) in enumerate(pairs):
        xi = sub_ref[:, _GCN * i:_GCN * (i + 1)]
        yj = sub_ref[:, _GCN * j:_GCN * (j + 1)]
        ypc = yp_ref[cnt * _B:(cnt + 1) * _B, :]
        h1 = jnp.maximum(
            jnp.dot(xi, w1_ref[cnt], precision=hp,
                    preferred_element_type=jnp.float32)
            + b1_ref[cnt:cnt + 1, :], 0.0)
        mu = (jnp.dot(h1, w2_ref[cnt], precision=hp,
                      preferred_element_type=jnp.float32)
              + b2_ref[cnt:cnt + 1, :])
        h2 = jnp.maximum(
            jnp.dot(xi, w3_ref[cnt], precision=hp,
                    preferred_element_type=jnp.float32)
            + b3_ref[cnt:cnt + 1, :], 0.0)
        logvar = jnp.tanh(
            jnp.dot(h2, w4_ref[cnt], precision=hp,
                    preferred_element_type=jnp.float32)
            + b4_ref[cnt:cnt + 1, :])
        inv_var = jnp.exp(-logvar)
        positive = -((mu - yj) ** 2) * inv_var
        negative = -((mu - ypc) ** 2) * inv_var
        upper_bound = (positive.sum(axis=-1) - negative.sum(axis=-1)).mean()
        mi = mi + upper_bound / 2.0
    out_ref[...] = mi.reshape(1, 1)


def kernel(init_embed, init_rel, w_mu1, b_mu1, w_mu2, b_mu2,
           w_lv1, b_lv1, w_lv2, b_lv2, sub, rel, obj):
    idx, dst = _build_indices(sub, rel, obj)

    sub_emb, obj_emb, yp, rel_emb = _make_sc_gather()(
        init_embed, init_rel, idx, dst)

    mi = pl.pallas_call(
        _mi_body,
        out_shape=jax.ShapeDtypeStruct((1, 1), jnp.float32),
    )(sub_emb, yp, w_mu1, b_mu1, w_mu2, b_mu2, w_lv1, b_lv1, w_lv2, b_lv2)
    mi_loss = mi[0, 0]

    return (sub_emb, rel_emb, obj_emb, init_embed, mi_loss)


# speedup vs baseline: 3.6088x; 1.9842x over previous
import functools

import jax
import jax.numpy as jnp
from jax import lax
from jax.experimental import pallas as pl
from jax.experimental.pallas import tpu as pltpu
from jax.experimental.pallas import tpu_sc as plsc

_NUM_ENT = 50000
_NF = 3
_GCN = 128
_DIM = _NF * _GCN
_B = 4096

_NC, _NS = 2, 16
_NW = _NC * _NS
_CH = _B // _NW

_J_OF = (1, 2, 2)


@functools.lru_cache(maxsize=1)
def _scatter_dst():
    import numpy as np
    base = np.arange(_B, dtype=np.int64)
    obj_dst = np.stack([3 * base + t for t in range(3)])
    with jax.ensure_compile_time_eval():
        pinv = [
            np.argsort(np.asarray(
                jax.random.permutation(
                    jax.random.fold_in(jax.random.key(123), c), _B)))
            for c in range(3)
        ]
    yp_dst = np.stack([t * _B + pinv[t] for t in range(3)])
    dst = np.concatenate(
        [
            obj_dst.reshape(3, _NW, _CH).transpose(1, 0, 2),
            yp_dst.reshape(3, _NW, _CH).transpose(1, 0, 2),
        ],
        axis=1,
    ).astype(np.int32)
    return dst


def _build_indices(sub, rel, obj):
    idx = jnp.stack(
        [sub.reshape(_NW, _CH), obj.reshape(_NW, _CH), rel.reshape(_NW, _CH)],
        axis=1,
    ).astype(jnp.int32)
    return idx, jnp.asarray(_scatter_dst())


def _sc_gather_body(emb, relt, idx, dst, out_sub, out_obj, out_yp, out_rel,
                    idx_v, dst_v, buf_sub, buf_obj, buf_rel,
                    sem_a, sem_b, sem_c):
    w = lax.axis_index("s") * _NC + lax.axis_index("c")
    pltpu.sync_copy(idx.at[w], idx_v)
    pltpu.sync_copy(dst.at[w], dst_v)

    h_sub = pltpu.async_copy(emb.at[idx_v.at[0]], buf_sub, sem_a)
    h_obj = pltpu.async_copy(emb.at[idx_v.at[1]], buf_obj, sem_b)
    h_rel = pltpu.async_copy(relt.at[idx_v.at[2]], buf_rel, sem_c)

    h_sub.wait()
    pltpu.sync_copy(buf_sub, out_sub.at[pl.ds(w * _CH, _CH)])
    for t in range(3):
        j = _J_OF[t]
        pltpu.async_copy(
            buf_sub.at[:, pl.ds(j * _GCN, _GCN)],
            out_yp.at[dst_v.at[3 + t]],
            sem_a,
        ).wait()

    h_obj.wait()
    for t in range(3):
        pltpu.async_copy(buf_obj, out_obj.at[dst_v.at[t]], sem_b).wait()

    h_rel.wait()
    for t in range(3):
        pltpu.sync_copy(
            buf_rel,
            out_rel.at[pl.ds(w * _CH, _CH), pl.ds(t * _GCN, _GCN)],
        )


@functools.lru_cache(maxsize=1)
def _make_sc_gather():
    return functools.partial(
        pl.kernel,
        out_type=(
            jax.ShapeDtypeStruct((_B, _DIM), jnp.float32),
            jax.ShapeDtypeStruct((_NF * _B, _DIM), jnp.float32),
            jax.ShapeDtypeStruct((3 * _B, _GCN), jnp.float32),
            jax.ShapeDtypeStruct((_B, _DIM), jnp.float32),
        ),
        mesh=plsc.VectorSubcoreMesh(
            core_axis_name="c", subcore_axis_name="s",
            num_cores=_NC, num_subcores=_NS,
        ),
        scratch_types=[
            pltpu.VMEM((3, _CH), jnp.int32),
            pltpu.VMEM((6, _CH), jnp.int32),
            pltpu.VMEM((_CH, _DIM), jnp.float32),
            pltpu.VMEM((_CH, _DIM), jnp.float32),
            pltpu.VMEM((_CH, _GCN), jnp.float32),
            pltpu.SemaphoreType.DMA,
            pltpu.SemaphoreType.DMA,
            pltpu.SemaphoreType.DMA,
        ],
    )(_sc_gather_body)


def _mi_body(sub_ref, yp_ref, w1_ref, b1_ref, w2_ref, b2_ref,
             w3_ref, b3_ref, w4_ref, b4_ref, out_ref):
    hp = None
    mi = jnp.float32(0.0)
    pairs = ((0, 1), (0, 2), (1, 2))
    for cnt, (i, j) in enumerate(pairs):
        xi = sub_ref[:, _GCN * i:_GCN * (i + 1)]
        yj = sub_ref[:, _GCN * j:_GCN * (j + 1)]
        ypc = yp_ref[cnt * _B:(cnt + 1) * _B, :]
        h1 = jnp.maximum(
            jnp.dot(xi, w1_ref[cnt], precision=hp,
                    preferred_element_type=jnp.float32)
            + b1_ref[cnt:cnt + 1, :], 0.0)
        mu = (jnp.dot(h1, w2_ref[cnt], precision=hp,
                      preferred_element_type=jnp.float32)
              + b2_ref[cnt:cnt + 1, :])
        h2 = jnp.maximum(
            jnp.dot(xi, w3_ref[cnt], precision=hp,
                    preferred_element_type=jnp.float32)
            + b3_ref[cnt:cnt + 1, :], 0.0)
        logvar = jnp.tanh(
            jnp.dot(h2, w4_ref[cnt], precision=hp,
                    preferred_element_type=jnp.float32)
            + b4_ref[cnt:cnt + 1, :])
        inv_var = jnp.exp(-logvar)
        positive = -((mu - yj) ** 2) * inv_var
        negative = -((mu - ypc) ** 2) * inv_var
        upper_bound = (positive.sum(axis=-1) - negative.sum(axis=-1)).mean()
        mi = mi + upper_bound / 2.0
    out_ref[...] = mi.reshape(1, 1)


def kernel(init_embed, init_rel, w_mu1, b_mu1, w_mu2, b_mu2,
           w_lv1, b_lv1, w_lv2, b_lv2, sub, rel, obj):
    idx, dst = _build_indices(sub, rel, obj)

    sub_emb, obj_emb, yp, rel_emb = _make_sc_gather()(
        init_embed, init_rel, idx, dst)

    mi = pl.pallas_call(
        _mi_body,
        out_shape=jax.ShapeDtypeStruct((1, 1), jnp.float32),
    )(sub_emb, yp, w_mu1, b_mu1, w_mu2, b_mu2, w_lv1, b_lv1, w_lv2, b_lv2)
    mi_loss = mi[0, 0]

    return (sub_emb, rel_emb, obj_emb, init_embed, mi_loss)
